# Initial kernel scaffold; baseline (speedup 1.0000x reference)
import jax
import jax.numpy as jnp
from jax.experimental import pallas as pl


def kernel(input_coords, tables):
    raise NotImplementedError("write your pallas kernel here")


# speedup vs baseline: 12.4383x; 12.4383x over previous
import jax
import jax.numpy as jnp
from jax import lax
from jax.experimental import pallas as pl
from jax.experimental.pallas import tpu as pltpu
from jax.experimental.pallas import tpu_sc as plsc

N_LEVELS = 16
N_MIN = 16.0
N_MAX = 512
HASH_EXP = 19
T = 2 ** HASH_EXP
MASK = T - 1
NUM_COORDS = 262144
PI2 = -1640531535

NC = 2
NS = 16
NW = NC * NS
NPW = NUM_COORDS // NW
C = 512
G = C // 16
NCHUNK = NPW // C


def _sc_body(x_hbm, y_hbm, sc_hbm, tbl_hbm, out_hbm,
             xv, yv, scv, idxv, subv, rowsv, rwv, cwv, outv, sem):
    wid = lax.axis_index("s") * jnp.int32(NC) + lax.axis_index("c")
    base = wid * jnp.int32(NPW)
    pltpu.sync_copy(sc_hbm, scv)
    lanes = lax.iota(jnp.int32, 16)

    def do_chunk(c, _):
        start = base + c * jnp.int32(C)
        pltpu.sync_copy(x_hbm.at[pl.ds(start, C)], xv)
        pltpu.sync_copy(y_hbm.at[pl.ds(start, C)], yv)

        for l in range(N_LEVELS):
            sv = scv[l, :]
            lvl_row = jnp.int32(l * T // 4)

            def pass1(g, _):
                gb = g * jnp.int32(16)
                xf = xv[pl.ds(gb, 16)]
                yf = yv[pl.ds(gb, 16)]
                sx = xf * sv
                sy = yf * sv
                nx0 = sx.astype(jnp.int32)
                nx1 = (sx + jnp.float32(1.0)).astype(jnp.int32)
                ny0 = sy.astype(jnp.int32)
                ny1 = (sy + jnp.float32(1.0)).astype(jnp.int32)
                rwv[pl.ds(gb, 16)] = sx - nx0.astype(jnp.float32)
                cwv[pl.ds(gb, 16)] = sy - ny0.astype(jnp.float32)
                hy0 = ny0 * jnp.int32(PI2)
                hy1 = ny1 * jnp.int32(PI2)
                mk = jnp.int32(MASK)
                two = jnp.int32(2)
                three = jnp.int32(3)
                i00 = (nx0 ^ hy0) & mk
                i01 = (nx0 ^ hy1) & mk
                i11 = (nx1 ^ hy1) & mk
                i10 = (nx1 ^ hy0) & mk
                for k, ii in enumerate((i00, i01, i11, i10)):
                    off = gb + jnp.int32(k * C)
                    idxv[pl.ds(off, 16)] = (
                        lax.shift_right_logical(ii, two) + lvl_row)
                    subv[pl.ds(off, 16)] = (ii & three) * two
                return _

            lax.fori_loop(jnp.int32(0), jnp.int32(G), pass1, None)

            pltpu.async_copy(tbl_hbm.at[idxv], rowsv, sem).wait()

            col0 = jnp.full((16,), 2 * l, jnp.int32)
            col1 = jnp.full((16,), 2 * l + 1, jnp.int32)

            def pass2(g, _):
                gb = g * jnp.int32(16)
                rw = rwv[pl.ds(gb, 16)]
                cw = cwv[pl.ds(gb, 16)]
                ridx = lanes + gb
                r00 = ridx
                r01 = ridx + jnp.int32(1 * C)
                r11 = ridx + jnp.int32(2 * C)
                r10 = ridx + jnp.int32(3 * C)
                s00 = subv[pl.ds(gb, 16)]
                s01 = subv[pl.ds(gb + jnp.int32(1 * C), 16)]
                s11 = subv[pl.ds(gb + jnp.int32(2 * C), 16)]
                s10 = subv[pl.ds(gb + jnp.int32(3 * C), 16)]
                a = jnp.float32(1.0) - cw
                b_ = jnp.float32(1.0) - rw
                for f, colc in ((0, col0), (1, col1)):
                    fo = jnp.int32(f)
                    v00 = plsc.load_gather(rowsv, [r00, s00 + fo])
                    v01 = plsc.load_gather(rowsv, [r01, s01 + fo])
                    v11 = plsc.load_gather(rowsv, [r11, s11 + fo])
                    v10 = plsc.load_gather(rowsv, [r10, s10 + fo])
                    o = (((v00 * a) * b_ + (v01 * cw) * b_)
                         + (v10 * a) * rw) + (v11 * cw) * rw
                    plsc.store_scatter(outv, [ridx, colc], o)
                return _

            lax.fori_loop(jnp.int32(0), jnp.int32(G), pass2, None)

        pltpu.sync_copy(outv, out_hbm.at[pl.ds(start, C)])
        return _

    lax.fori_loop(jnp.int32(0), jnp.int32(NCHUNK), do_chunk, None)


@jax.jit
def _run(xf, yf, scales_b, tbl8):
    mesh = plsc.VectorSubcoreMesh(core_axis_name="c", subcore_axis_name="s")
    k = pl.kernel(
        _sc_body,
        out_type=jax.ShapeDtypeStruct((NUM_COORDS, 2 * N_LEVELS), jnp.float32),
        mesh=mesh,
        compiler_params=pltpu.CompilerParams(
            needs_layout_passes=False, use_tc_tiling_on_sc=False),
        scratch_types=[
            pltpu.VMEM((C,), jnp.float32),
            pltpu.VMEM((C,), jnp.float32),
            pltpu.VMEM((N_LEVELS, 16), jnp.float32),
            pltpu.VMEM((4 * C,), jnp.int32),
            pltpu.VMEM((4 * C,), jnp.int32),
            pltpu.VMEM((4 * C, 8), jnp.float32),
            pltpu.VMEM((C,), jnp.float32),
            pltpu.VMEM((C,), jnp.float32),
            pltpu.VMEM((C, 2 * N_LEVELS), jnp.float32),
            pltpu.SemaphoreType.DMA,
        ],
    )
    return k(xf, yf, scales_b, tbl8)


def kernel(input_coords, tables):
    coords_f = input_coords.astype(jnp.float32)
    xf = coords_f[:, 0]
    yf = coords_f[:, 1]
    b = jnp.exp((jnp.log(jnp.float32(N_MAX)) - jnp.log(jnp.float32(N_MIN)))
                / (N_LEVELS - 1))
    scales = jnp.stack(
        [jnp.floor(jnp.float32(N_MIN) * b ** i) / jnp.float32(N_MAX)
         for i in range(N_LEVELS)])
    scales_b = jnp.broadcast_to(scales[:, None], (N_LEVELS, 16))
    tbl8 = tables.reshape(N_LEVELS * T // 4, 8)
    return _run(xf, yf, scales_b, tbl8)
